# baseline (speedup 1.0000x reference)
import functools

import jax
import jax.numpy as jnp
from jax import lax
from jax.experimental import pallas as pl
from jax.experimental.pallas import tpu as pltpu
from jax.experimental.pallas import tpu_sc as plsc

_ALPHA = 0.001
_NC = 2
_NS = 16
_CH = 128


def _build(B, K, n_m, n_d):
    nw = _NC * _NS
    b_per_w = B // nw
    n_chunks = b_per_w // _CH
    mesh = plsc.VectorSubcoreMesh(core_axis_name="c", subcore_axis_name="s")

    @functools.partial(
        pl.kernel,
        mesh=mesh,
        compiler_params=pltpu.CompilerParams(use_tc_tiling_on_sc=False),
        out_type=jax.ShapeDtypeStruct((B,), jnp.float32),
        scratch_types=[
            pltpu.VMEM((n_chunks, _CH), jnp.int32),
            pltpu.VMEM((n_chunks, _CH), jnp.int32),
            pltpu.VMEM((n_chunks, K * _CH), jnp.int32),
            pltpu.VMEM((n_chunks, K * _CH), jnp.int32),
            pltpu.VMEM((n_chunks, K * _CH), jnp.float32),
            pltpu.VMEM((n_chunks, K * _CH), jnp.float32),
            pltpu.VMEM((b_per_w,), jnp.float32),
            pltpu.VMEM((b_per_w,), jnp.float32),
            pltpu.VMEM((b_per_w,), jnp.float32),
            pltpu.SemaphoreType.DMA,
            pltpu.SemaphoreType.DMA,
        ],
    )
    def sc_kernel(i_hbm, j_hbm, mbar_hbm, dbar_hbm, mflat_hbm, dflat_hbm,
                  out_hbm, i_v, j_v, mi_v, di_v, mg, dg, mbv, dbv, outv,
                  sem_rows, sem_bar):
        wid = lax.axis_index("s") * _NC + lax.axis_index("c")
        base = wid * b_per_w

        pltpu.sync_copy(i_hbm.at[wid], i_v)
        pltpu.sync_copy(j_hbm.at[wid], j_v)

        for c in range(n_chunks):
            ic = i_v[c]
            jc = j_v[c]
            for k in range(K):
                sl = pl.ds(k * _CH, _CH)
                mi_v[c, sl] = ic + k * n_m
                di_v[c, sl] = jc + k * n_d

        copies = []
        for c in range(n_chunks):
            sl = pl.ds(c * _CH, _CH)
            copies.append(
                pltpu.async_copy(mflat_hbm.at[mi_v.at[c]], mg.at[c], sem_rows))
            copies.append(
                pltpu.async_copy(dflat_hbm.at[di_v.at[c]], dg.at[c], sem_rows))
            copies.append(
                pltpu.async_copy(mbar_hbm.at[i_v.at[c]], mbv.at[sl], sem_bar))
            copies.append(
                pltpu.async_copy(dbar_hbm.at[j_v.at[c]], dbv.at[sl], sem_bar))
        for cp in copies:
            cp.wait()

        for c in range(n_chunks):
            acc = mg[c, pl.ds(0, _CH)] * dg[c, pl.ds(0, _CH)]
            for k in range(1, K):
                sl = pl.ds(k * _CH, _CH)
                acc = acc + mg[c, sl] * dg[c, sl]
            out_sl = pl.ds(c * _CH, _CH)
            outv[out_sl] = mbv[out_sl] + dbv[out_sl] + _ALPHA * acc

        pltpu.sync_copy(outv, out_hbm.at[pl.ds(base, b_per_w)])

    return sc_kernel


@jax.jit
def _run(ij, m_bar, d_bar, M, D):
    B = ij.shape[0]
    n_m, K = M.shape
    n_d = D.shape[0]
    nw = _NC * _NS
    i_idx = ij[:, 0].reshape(nw, -1, _CH)
    j_idx = ij[:, 1].reshape(nw, -1, _CH)
    m_flat = M.T.reshape(-1)
    d_flat = D.T.reshape(-1)
    return _build(B, K, n_m, n_d)(i_idx, j_idx, m_bar, d_bar, m_flat, d_flat)


def kernel(ij, m_bar, d_bar, M, D):
    return _run(ij, m_bar, d_bar, M, D)
